# baseline (speedup 1.0000x reference)
import functools

import jax
import jax.numpy as jnp
from jax import lax
from jax.experimental import pallas as pl
from jax.experimental.pallas import tpu as pltpu
from jax.experimental.pallas import tpu_sc as plsc

B = 500
NUMK = 100
EPER = 1600
BPAD = 512
BB = 64
NTILES = 32
SPT = BPAD // NTILES


SDIM = NUMK * NUMK


def _make_sc_build(npart, h):
    spt = npart // NTILES
    goff = h * npart

    def _sc_build(ew_hbm, ei_hbm, z_hbm, s_hbm,
                  row_v, col_v, ew0, ew1, ew2, sv0, sv1,
                  es0, es1, es2, ss0, ss1):
        wid = lax.axis_index("s") * 2 + lax.axis_index("c")
        pltpu.sync_copy(ei_hbm.at[0], row_v)
        pltpu.sync_copy(ei_hbm.at[1], col_v)
        pltpu.sync_copy(z_hbm, sv0)
        pltpu.sync_copy(z_hbm, sv1)

        ewb = [ew0, ew1, ew2]
        esem = [es0, es1, es2]
        svb = [sv0, sv1]
        ssem = [ss0, ss1]
        base = wid * spt

        def scat_pass(s_v, ew_v, sign):
            def scat(k, cc):
                for u in range(10):
                    sl = pl.ds((k * 10 + u) * 16, 16)
                    plsc.addupdate_scatter(
                        s_v, [col_v[sl], row_v[sl]], sign * ew_v[sl])
                return cc
            lax.fori_loop(0, EPER // 160, scat, 0)

        def prefetch(i):
            if i < spt:
                @pl.when(goff + base + i < B)
                def _():
                    pltpu.async_copy(ew_hbm.at[goff + base + i],
                                     ewb[i % 3], esem[i % 3])

        prefetch(0)
        for i in range(spt):
            b = base + i
            p = i % 2

            if i >= 2:
                @pl.when(goff + base + i - 2 < B)
                def _(i=i, p=p):
                    pltpu.make_async_copy(svb[p], s_hbm.at[base + i - 2],
                                          ssem[p]).wait()
                    scat_pass(svb[p], ewb[(i - 2) % 3], -1.0)

            @pl.when(goff + b < B)
            def _(i=i, p=p, b=b):
                pltpu.make_async_copy(ew_hbm.at[goff + b], ewb[i % 3],
                                      esem[i % 3]).wait()
                scat_pass(svb[p], ewb[i % 3], 1.0)
                pltpu.async_copy(svb[p], s_hbm.at[b], ssem[p])
            prefetch(i + 1)

        for i in range(spt - 2, spt):
            @pl.when(goff + base + i < B)
            def _(i=i):
                pltpu.make_async_copy(svb[i % 2], s_hbm.at[base + i],
                                      ssem[i % 2]).wait()
    return _sc_build


def _build_s(ew, edge_index, npart, h):
    mesh = plsc.VectorSubcoreMesh(core_axis_name="c", subcore_axis_name="s")
    f = pl.kernel(
        _make_sc_build(npart, h),
        mesh=mesh,
        compiler_params=pltpu.CompilerParams(needs_layout_passes=False),
        out_type=jax.ShapeDtypeStruct((npart, NUMK, NUMK), jnp.float32),
        scratch_types=[
            pltpu.VMEM((EPER,), jnp.int32),
            pltpu.VMEM((EPER,), jnp.int32),
            pltpu.VMEM((EPER,), jnp.float32),
            pltpu.VMEM((EPER,), jnp.float32),
            pltpu.VMEM((EPER,), jnp.float32),
            pltpu.VMEM((NUMK, NUMK), jnp.float32),
            pltpu.VMEM((NUMK, NUMK), jnp.float32),
            pltpu.SemaphoreType.DMA,
            pltpu.SemaphoreType.DMA,
            pltpu.SemaphoreType.DMA,
            pltpu.SemaphoreType.DMA,
            pltpu.SemaphoreType.DMA,
        ],
    )
    zeros = jnp.zeros((NUMK, NUMK), jnp.float32)
    return f(ew, edge_index, zeros)


def _gcn_body(s_ref, p_ref, w1_ref, b1_ref, w2_ref, b2_ref, w3_ref, b3_ref,
              out_ref):
    f32 = jnp.float32
    eye = (jax.lax.broadcasted_iota(jnp.int32, (NUMK, NUMK), 0)
           == jax.lax.broadcasted_iota(jnp.int32, (NUMK, NUMK), 1)).astype(f32)

    w1 = w1_ref[...]
    b1 = b1_ref[...]
    w2 = w2_ref[...]
    b2 = b2_ref[...]
    w3 = w3_ref[...]
    b3 = b3_ref[...]

    R = range(BB)
    spi = [s_ref[b] + eye for b in R]
    deg = [jnp.sum(spi[b], axis=1, keepdims=True) for b in R]
    dis = [jnp.where(deg[b] > 0, jax.lax.rsqrt(deg[b]), 0.0) for b in R]
    x0 = [jnp.transpose(p_ref[pl.ds(b, 1), :]) for b in R]
    q1 = [dis[b] * jnp.dot(spi[b], dis[b] * x0[b], preferred_element_type=f32)
          for b in R]
    h1 = [jnp.maximum(q1[b] * w1 + b1, 0.0) for b in R]
    t2 = [jnp.dot(h1[b], w2, preferred_element_type=f32) for b in R]
    q2 = [dis[b] * jnp.dot(spi[b], dis[b] * t2[b], preferred_element_type=f32)
          for b in R]
    h2 = [jnp.maximum(q2[b] + b2, 0.0) for b in R]
    t3 = [jnp.dot(h2[b], w3, preferred_element_type=f32) for b in R]
    q3 = [dis[b] * jnp.dot(spi[b], dis[b] * t3[b], preferred_element_type=f32)
          for b in R]
    rows = [jnp.transpose(q3[b] + b3) for b in R]
    out_ref[...] = jnp.concatenate(rows, axis=0)


HALF = BPAD // 2


def _tc_consume(s3, p, h, W1, b1, W2, b2, W3, b3):
    f32 = jnp.float32
    npart = s3.shape[0]
    hoff = h * (npart // BB)
    grid = (npart // BB,)
    return pl.pallas_call(
        _gcn_body,
        grid=grid,
        in_specs=[
            pl.BlockSpec((BB, NUMK, NUMK), lambda i: (i, 0, 0)),
            pl.BlockSpec((BB, NUMK), lambda i: (i + hoff, 0)),
            pl.BlockSpec((1, 32), lambda i: (0, 0)),
            pl.BlockSpec((1, 32), lambda i: (0, 0)),
            pl.BlockSpec((32, 32), lambda i: (0, 0)),
            pl.BlockSpec((1, 32), lambda i: (0, 0)),
            pl.BlockSpec((32, 1), lambda i: (0, 0)),
            pl.BlockSpec((1, 1), lambda i: (0, 0)),
        ],
        out_specs=pl.BlockSpec((BB, NUMK), lambda i: (i, 0)),
        out_shape=jax.ShapeDtypeStruct((npart, NUMK), f32),
    )(s3, p,
      W1, b1.reshape(1, 32), W2, b2.reshape(1, 32), W3, b3.reshape(1, 1))


@jax.jit
def kernel(Hx, edge_index, W1, b1, W2, b2, W3, b3):
    f32 = jnp.float32
    p = jnp.zeros((BPAD, NUMK), f32).at[:B].set(Hx[:, :NUMK])
    ew = Hx[:, NUMK:NUMK + EPER]

    outs = []
    for h in range(2):
        s3 = _build_s(ew, edge_index, HALF, h)
        outs.append(_tc_consume(s3, p, h, W1, b1, W2, b2, W3, b3))
    return jnp.concatenate(outs, axis=0)[:B]


# speedup vs baseline: 1.0160x; 1.0160x over previous
import functools

import jax
import jax.numpy as jnp
from jax import lax
from jax.experimental import pallas as pl
from jax.experimental.pallas import tpu as pltpu
from jax.experimental.pallas import tpu_sc as plsc

B = 500
NUMK = 100
EPER = 1600
BPAD = 512
BB = 64
NTILES = 32
SPT = BPAD // NTILES


SDIM = NUMK * NUMK


def _make_sc_build(npart, h):
    spt = npart // NTILES
    goff = h * npart

    def _sc_build(ew_hbm, ei_hbm, z_hbm, s_hbm,
                  row_v, col_v, ew0, ew1, ew2, sv0, sv1,
                  es0, es1, es2, ss0, ss1):
        wid = lax.axis_index("s") * 2 + lax.axis_index("c")

        ewb = [ew0, ew1, ew2]
        esem = [es0, es1, es2]
        svb = [sv0, sv1]
        ssem = [ss0, ss1]
        base = wid * spt

        c1 = pltpu.async_copy(ei_hbm.at[0], row_v, es2)
        c2 = pltpu.async_copy(ei_hbm.at[1], col_v, ss0)
        c3 = pltpu.async_copy(z_hbm, sv0, ss1)
        c4 = pltpu.async_copy(z_hbm, sv1, es1)

        def scat_pass(s_v, ew_v, sign):
            def scat(k, cc):
                for u in range(10):
                    sl = pl.ds((k * 10 + u) * 16, 16)
                    plsc.addupdate_scatter(
                        s_v, [col_v[sl], row_v[sl]], sign * ew_v[sl])
                return cc
            lax.fori_loop(0, EPER // 160, scat, 0)

        def prefetch(i):
            if i < spt:
                @pl.when(goff + base + i < B)
                def _():
                    pltpu.async_copy(ew_hbm.at[goff + base + i],
                                     ewb[i % 3], esem[i % 3])

        prefetch(0)
        c1.wait()
        c2.wait()
        c3.wait()
        c4.wait()
        for i in range(spt):
            b = base + i
            p = i % 2

            if i >= 2:
                @pl.when(goff + base + i - 2 < B)
                def _(i=i, p=p):
                    pltpu.make_async_copy(svb[p], s_hbm.at[base + i - 2],
                                          ssem[p]).wait()
                    scat_pass(svb[p], ewb[(i - 2) % 3], -1.0)

            @pl.when(goff + b < B)
            def _(i=i, p=p, b=b):
                pltpu.make_async_copy(ew_hbm.at[goff + b], ewb[i % 3],
                                      esem[i % 3]).wait()
                scat_pass(svb[p], ewb[i % 3], 1.0)
                pltpu.async_copy(svb[p], s_hbm.at[b], ssem[p])
            prefetch(i + 1)

        for i in range(spt - 2, spt):
            @pl.when(goff + base + i < B)
            def _(i=i):
                pltpu.make_async_copy(svb[i % 2], s_hbm.at[base + i],
                                      ssem[i % 2]).wait()
    return _sc_build


def _build_s(ew, edge_index, npart, h):
    mesh = plsc.VectorSubcoreMesh(core_axis_name="c", subcore_axis_name="s")
    f = pl.kernel(
        _make_sc_build(npart, h),
        mesh=mesh,
        compiler_params=pltpu.CompilerParams(needs_layout_passes=False),
        out_type=jax.ShapeDtypeStruct((npart, NUMK, NUMK), jnp.float32),
        scratch_types=[
            pltpu.VMEM((EPER,), jnp.int32),
            pltpu.VMEM((EPER,), jnp.int32),
            pltpu.VMEM((EPER,), jnp.float32),
            pltpu.VMEM((EPER,), jnp.float32),
            pltpu.VMEM((EPER,), jnp.float32),
            pltpu.VMEM((NUMK, NUMK), jnp.float32),
            pltpu.VMEM((NUMK, NUMK), jnp.float32),
            pltpu.SemaphoreType.DMA,
            pltpu.SemaphoreType.DMA,
            pltpu.SemaphoreType.DMA,
            pltpu.SemaphoreType.DMA,
            pltpu.SemaphoreType.DMA,
        ],
    )
    zeros = jnp.zeros((NUMK, NUMK), jnp.float32)
    return f(ew, edge_index, zeros)


def _gcn_body(s_ref, p_ref, w1_ref, b1_ref, w2_ref, b2_ref, w3_ref, b3_ref,
              out_ref):
    f32 = jnp.float32
    eye = (jax.lax.broadcasted_iota(jnp.int32, (NUMK, NUMK), 0)
           == jax.lax.broadcasted_iota(jnp.int32, (NUMK, NUMK), 1)).astype(f32)

    w1 = w1_ref[...]
    b1 = b1_ref[...]
    w2 = w2_ref[...]
    b2 = b2_ref[...]
    w3 = w3_ref[...]
    b3 = b3_ref[...]

    R = range(BB)
    spi = [s_ref[b] + eye for b in R]
    deg = [jnp.sum(spi[b], axis=1, keepdims=True) for b in R]
    dis = [jnp.where(deg[b] > 0, jax.lax.rsqrt(deg[b]), 0.0) for b in R]
    x0 = [jnp.transpose(p_ref[pl.ds(b, 1), :]) for b in R]
    q1 = [dis[b] * jnp.dot(spi[b], dis[b] * x0[b], preferred_element_type=f32)
          for b in R]
    h1 = [jnp.maximum(q1[b] * w1 + b1, 0.0) for b in R]
    t2 = [jnp.dot(h1[b], w2, preferred_element_type=f32) for b in R]
    q2 = [dis[b] * jnp.dot(spi[b], dis[b] * t2[b], preferred_element_type=f32)
          for b in R]
    h2 = [jnp.maximum(q2[b] + b2, 0.0) for b in R]
    t3 = [jnp.dot(h2[b], w3, preferred_element_type=f32) for b in R]
    q3 = [dis[b] * jnp.dot(spi[b], dis[b] * t3[b], preferred_element_type=f32)
          for b in R]
    rows = [jnp.transpose(q3[b] + b3) for b in R]
    out_ref[...] = jnp.concatenate(rows, axis=0)


HALF = BPAD // 2


def _tc_consume(s3, p, h, W1, b1, W2, b2, W3, b3):
    f32 = jnp.float32
    npart = s3.shape[0]
    hoff = h * (npart // BB)
    grid = (npart // BB,)
    return pl.pallas_call(
        _gcn_body,
        grid=grid,
        in_specs=[
            pl.BlockSpec((BB, NUMK, NUMK), lambda i: (i, 0, 0)),
            pl.BlockSpec((BB, NUMK), lambda i: (i + hoff, 0)),
            pl.BlockSpec((1, 32), lambda i: (0, 0)),
            pl.BlockSpec((1, 32), lambda i: (0, 0)),
            pl.BlockSpec((32, 32), lambda i: (0, 0)),
            pl.BlockSpec((1, 32), lambda i: (0, 0)),
            pl.BlockSpec((32, 1), lambda i: (0, 0)),
            pl.BlockSpec((1, 1), lambda i: (0, 0)),
        ],
        out_specs=pl.BlockSpec((BB, NUMK), lambda i: (i, 0)),
        out_shape=jax.ShapeDtypeStruct((npart, NUMK), f32),
    )(s3, p,
      W1, b1.reshape(1, 32), W2, b2.reshape(1, 32), W3, b3.reshape(1, 1))


@jax.jit
def kernel(Hx, edge_index, W1, b1, W2, b2, W3, b3):
    f32 = jnp.float32
    p = jnp.zeros((BPAD, NUMK), f32).at[:B].set(Hx[:, :NUMK])
    ew = Hx[:, NUMK:NUMK + EPER]

    outs = []
    for h in range(2):
        s3 = _build_s(ew, edge_index, HALF, h)
        outs.append(_tc_consume(s3, p, h, W1, b1, W2, b2, W3, b3))
    return jnp.concatenate(outs, axis=0)[:B]


# speedup vs baseline: 1.0170x; 1.0010x over previous
import jax
import jax.numpy as jnp
from jax import lax
from jax.experimental import pallas as pl
from jax.experimental.pallas import tpu as pltpu
from jax.experimental.pallas import tpu_sc as plsc

B = 500
NUMK = 100
EPER = 1600
BPAD = 512
BB = 64
NTILES = 32
SPT = BPAD // NTILES


SDIM = NUMK * NUMK


def _make_sc_build(npart, h):
    spt = npart // NTILES
    goff = h * npart

    def _sc_build(ew_hbm, ei_hbm, z_hbm, s_hbm,
                  row_v, col_v, ew0, ew1, ew2, sv0, sv1,
                  es0, es1, es2, ss0, ss1):
        wid = lax.axis_index("s") * 2 + lax.axis_index("c")

        ewb = [ew0, ew1, ew2]
        esem = [es0, es1, es2]
        svb = [sv0, sv1]
        ssem = [ss0, ss1]
        base = wid * spt

        c1 = pltpu.async_copy(ei_hbm.at[0], row_v, es2)
        c2 = pltpu.async_copy(ei_hbm.at[1], col_v, ss0)
        c3 = pltpu.async_copy(z_hbm, sv0, ss1)
        c4 = pltpu.async_copy(z_hbm, sv1, es1)

        def scat_pass(s_v, ew_v, sign):
            def scat(k, cc):
                for u in range(10):
                    sl = pl.ds((k * 10 + u) * 16, 16)
                    plsc.addupdate_scatter(
                        s_v, [col_v[sl], row_v[sl]], sign * ew_v[sl])
                return cc
            lax.fori_loop(0, EPER // 160, scat, 0)

        def prefetch(i):
            if i < spt:
                @pl.when(goff + base + i < B)
                def _():
                    pltpu.async_copy(ew_hbm.at[goff + base + i],
                                     ewb[i % 3], esem[i % 3])

        prefetch(0)
        c1.wait()
        c2.wait()
        c3.wait()
        c4.wait()
        for i in range(spt):
            b = base + i
            p = i % 2

            if i >= 2:
                @pl.when(goff + base + i - 2 < B)
                def _(i=i, p=p):
                    pltpu.make_async_copy(svb[p], s_hbm.at[base + i - 2],
                                          ssem[p]).wait()
                    scat_pass(svb[p], ewb[(i - 2) % 3], -1.0)

            @pl.when(goff + b < B)
            def _(i=i, p=p, b=b):
                pltpu.make_async_copy(ew_hbm.at[goff + b], ewb[i % 3],
                                      esem[i % 3]).wait()
                scat_pass(svb[p], ewb[i % 3], 1.0)
                pltpu.async_copy(svb[p], s_hbm.at[b], ssem[p])
            prefetch(i + 1)

        for i in range(spt - 2, spt):
            @pl.when(goff + base + i < B)
            def _(i=i):
                pltpu.make_async_copy(svb[i % 2], s_hbm.at[base + i],
                                      ssem[i % 2]).wait()
    return _sc_build


def _build_s(ew, edge_index, npart, h):
    mesh = plsc.VectorSubcoreMesh(core_axis_name="c", subcore_axis_name="s")
    f = pl.kernel(
        _make_sc_build(npart, h),
        mesh=mesh,
        compiler_params=pltpu.CompilerParams(needs_layout_passes=False),
        out_type=jax.ShapeDtypeStruct((npart, NUMK, NUMK), jnp.float32),
        scratch_types=[
            pltpu.VMEM((EPER,), jnp.int32),
            pltpu.VMEM((EPER,), jnp.int32),
            pltpu.VMEM((EPER,), jnp.float32),
            pltpu.VMEM((EPER,), jnp.float32),
            pltpu.VMEM((EPER,), jnp.float32),
            pltpu.VMEM((NUMK, NUMK), jnp.float32),
            pltpu.VMEM((NUMK, NUMK), jnp.float32),
            pltpu.SemaphoreType.DMA,
            pltpu.SemaphoreType.DMA,
            pltpu.SemaphoreType.DMA,
            pltpu.SemaphoreType.DMA,
            pltpu.SemaphoreType.DMA,
        ],
    )
    zeros = jnp.zeros((NUMK, NUMK), jnp.float32)
    return f(ew, edge_index, zeros)


def _gcn_body(s_ref, p_ref, w1_ref, b1_ref, w2_ref, b2_ref, w3_ref, b3_ref,
              out_ref):
    f32 = jnp.float32
    eye = (jax.lax.broadcasted_iota(jnp.int32, (NUMK, NUMK), 0)
           == jax.lax.broadcasted_iota(jnp.int32, (NUMK, NUMK), 1)).astype(f32)

    w1 = w1_ref[...]
    b1 = b1_ref[...]
    w2 = w2_ref[...]
    b2 = b2_ref[...]
    w3 = w3_ref[...]
    b3 = b3_ref[...]

    R = range(BB)
    spi = [s_ref[b] + eye for b in R]
    deg = [jnp.sum(spi[b], axis=1, keepdims=True) for b in R]
    dis = [jnp.where(deg[b] > 0, jax.lax.rsqrt(deg[b]), 0.0) for b in R]
    x0 = [jnp.transpose(p_ref[pl.ds(b, 1), :]) for b in R]
    q1 = [dis[b] * jnp.dot(spi[b], dis[b] * x0[b], preferred_element_type=f32)
          for b in R]
    h1 = [jnp.maximum(q1[b] * w1 + b1, 0.0) for b in R]
    t2 = [jnp.dot(h1[b], w2, preferred_element_type=f32) for b in R]
    q2 = [dis[b] * jnp.dot(spi[b], dis[b] * t2[b], preferred_element_type=f32)
          for b in R]
    h2 = [jnp.maximum(q2[b] + b2, 0.0) for b in R]
    t3 = [jnp.dot(h2[b], w3, preferred_element_type=f32) for b in R]
    q3 = [dis[b] * jnp.dot(spi[b], dis[b] * t3[b], preferred_element_type=f32)
          for b in R]
    rows = [jnp.transpose(q3[b] + b3) for b in R]
    out_ref[...] = jnp.concatenate(rows, axis=0)


HALF = BPAD // 2


def _tc_consume(s3, p, h, W1, b1, W2, b2, W3, b3):
    f32 = jnp.float32
    npart = s3.shape[0]
    hoff = h * (npart // BB)
    grid = (npart // BB,)
    return pl.pallas_call(
        _gcn_body,
        grid=grid,
        in_specs=[
            pl.BlockSpec((BB, NUMK, NUMK), lambda i: (i, 0, 0)),
            pl.BlockSpec((BB, NUMK), lambda i: (i + hoff, 0)),
            pl.BlockSpec((1, 32), lambda i: (0, 0)),
            pl.BlockSpec((1, 32), lambda i: (0, 0)),
            pl.BlockSpec((32, 32), lambda i: (0, 0)),
            pl.BlockSpec((1, 32), lambda i: (0, 0)),
            pl.BlockSpec((32, 1), lambda i: (0, 0)),
            pl.BlockSpec((1, 1), lambda i: (0, 0)),
        ],
        out_specs=pl.BlockSpec((BB, NUMK), lambda i: (i, 0)),
        out_shape=jax.ShapeDtypeStruct((npart, NUMK), f32),
    )(s3, p,
      W1, b1.reshape(1, 32), W2, b2.reshape(1, 32), W3, b3.reshape(1, 1))


@jax.jit
def kernel(Hx, edge_index, W1, b1, W2, b2, W3, b3):
    f32 = jnp.float32
    p = jnp.zeros((BPAD, NUMK), f32).at[:B].set(Hx[:, :NUMK])
    ew = Hx[:, NUMK:NUMK + EPER]

    outs = []
    for h in range(2):
        s3 = _build_s(ew, edge_index, HALF, h)
        outs.append(_tc_consume(s3, p, h, W1, b1, W2, b2, W3, b3))
    return jnp.concatenate(outs, axis=0)[:B]
